# baseline (speedup 1.0000x reference)
import functools

import jax
import jax.numpy as jnp
from jax import lax
from jax.experimental import pallas as pl
from jax.experimental.pallas import tpu as pltpu
from jax.experimental.pallas import tpu_sc as plsc

EMBED_DIM = 64
BATCH = 16384
N_FIELDS = 26
TOTAL = BATCH * N_FIELDS

NUM_CORES = 2
NUM_SUBCORES = 16
NW = NUM_CORES * NUM_SUBCORES
PER_W = TOTAL // NW
CHUNK = 128
N_CHUNKS = PER_W // CHUNK

_mesh = plsc.VectorSubcoreMesh(core_axis_name="c", subcore_axis_name="s")


@functools.partial(
    pl.kernel,
    mesh=_mesh,
    out_type=jax.ShapeDtypeStruct((TOTAL, EMBED_DIM), jnp.float32),
    scratch_types=[
        pltpu.VMEM((N_CHUNKS, CHUNK), jnp.int32),
        pltpu.VMEM((CHUNK, EMBED_DIM), jnp.float32),
        pltpu.SemaphoreType.DMA,
    ],
    compiler_params=pltpu.CompilerParams(use_tc_tiling_on_sc=False),
)
def _gather_k(idx_hbm, table_hbm, out_hbm, idx_v, buf_v, gsem):
    wid = lax.axis_index("s") * NUM_CORES + lax.axis_index("c")
    base = wid * PER_W
    pltpu.sync_copy(idx_hbm.at[wid], idx_v)

    def body(i, carry):
        pltpu.async_copy(table_hbm.at[idx_v.at[i]], buf_v, gsem).wait()
        pltpu.sync_copy(buf_v, out_hbm.at[pl.ds(base + i * CHUNK, CHUNK)])
        return carry

    lax.fori_loop(0, N_CHUNKS, body, 0)


def kernel(index, cluster_index):
    idx = index.reshape(NW, N_CHUNKS, CHUNK)
    out = _gather_k(idx, cluster_index)
    return out.reshape(BATCH, N_FIELDS, EMBED_DIM)


# speedup vs baseline: 1.0636x; 1.0636x over previous
import functools

import jax
import jax.numpy as jnp
from jax import lax
from jax.experimental import pallas as pl
from jax.experimental.pallas import tpu as pltpu
from jax.experimental.pallas import tpu_sc as plsc

EMBED_DIM = 64
BATCH = 16384
N_FIELDS = 26
TOTAL = BATCH * N_FIELDS

NUM_CORES = 2
NUM_SUBCORES = 16
NW = NUM_CORES * NUM_SUBCORES
PER_W = TOTAL // NW
CHUNK = 512
N_CHUNKS = PER_W // CHUNK

_mesh = plsc.VectorSubcoreMesh(core_axis_name="c", subcore_axis_name="s")


@functools.partial(
    pl.kernel,
    mesh=_mesh,
    out_type=jax.ShapeDtypeStruct((TOTAL, EMBED_DIM), jnp.float32),
    scratch_types=[
        pltpu.VMEM((N_CHUNKS, CHUNK), jnp.int32),
        pltpu.VMEM((CHUNK, EMBED_DIM), jnp.float32),
        pltpu.SemaphoreType.DMA,
    ],
    compiler_params=pltpu.CompilerParams(use_tc_tiling_on_sc=False),
)
def _gather_k(idx_hbm, table_hbm, out_hbm, idx_v, buf_v, gsem):
    wid = lax.axis_index("s") * NUM_CORES + lax.axis_index("c")
    base = wid * PER_W
    pltpu.sync_copy(idx_hbm.at[wid], idx_v)

    def body(i, carry):
        pltpu.async_copy(table_hbm.at[idx_v.at[i]], buf_v, gsem).wait()
        pltpu.sync_copy(buf_v, out_hbm.at[pl.ds(base + i * CHUNK, CHUNK)])
        return carry

    lax.fori_loop(0, N_CHUNKS, body, 0)


def kernel(index, cluster_index):
    idx = index.reshape(NW, N_CHUNKS, CHUNK)
    out = _gather_k(idx, cluster_index)
    return out.reshape(BATCH, N_FIELDS, EMBED_DIM)


# speedup vs baseline: 1.0785x; 1.0140x over previous
import functools

import jax
import jax.numpy as jnp
from jax import lax
from jax.experimental import pallas as pl
from jax.experimental.pallas import tpu as pltpu
from jax.experimental.pallas import tpu_sc as plsc

EMBED_DIM = 64
BATCH = 16384
N_FIELDS = 26
TOTAL = BATCH * N_FIELDS

NUM_CORES = 2
NUM_SUBCORES = 16
NW = NUM_CORES * NUM_SUBCORES
PER_W = TOTAL // NW
CHUNK = 512
N_CHUNKS = PER_W // CHUNK

_mesh = plsc.VectorSubcoreMesh(core_axis_name="c", subcore_axis_name="s")


@functools.partial(
    pl.kernel,
    mesh=_mesh,
    out_type=jax.ShapeDtypeStruct((TOTAL, EMBED_DIM), jnp.float32),
    scratch_types=[
        pltpu.VMEM((N_CHUNKS, CHUNK), jnp.int32),
        pltpu.VMEM((CHUNK, EMBED_DIM), jnp.float32),
        pltpu.VMEM((CHUNK, EMBED_DIM), jnp.float32),
        pltpu.SemaphoreType.DMA,
        pltpu.SemaphoreType.DMA,
        pltpu.SemaphoreType.DMA,
        pltpu.SemaphoreType.DMA,
    ],
    compiler_params=pltpu.CompilerParams(use_tc_tiling_on_sc=False),
)
def _gather_k(idx_hbm, table_hbm, out_hbm, idx_v, buf0, buf1, g0, g1, s0, s1):
    wid = lax.axis_index("s") * NUM_CORES + lax.axis_index("c")
    base = wid * PER_W
    pltpu.sync_copy(idx_hbm.at[wid], idx_v)

    bufs = (buf0, buf1)
    gsems = (g0, g1)
    ssems = (s0, s1)

    pltpu.async_copy(table_hbm.at[idx_v.at[0]], buf0, g0)
    pltpu.async_copy(table_hbm.at[idx_v.at[1]], buf1, g1)

    def body(i, carry):
        for p in range(2):
            c = i * 2 + p
            buf, gsem, ssem = bufs[p], gsems[p], ssems[p]
            pltpu.make_async_copy(table_hbm.at[idx_v.at[c]], buf, gsem).wait()
            pltpu.async_copy(buf, out_hbm.at[pl.ds(base + c * CHUNK, CHUNK)], ssem)

            @pl.when(c + 2 < N_CHUNKS)
            def _():
                pltpu.make_async_copy(
                    buf, out_hbm.at[pl.ds(base + c * CHUNK, CHUNK)], ssem
                ).wait()
                pltpu.async_copy(table_hbm.at[idx_v.at[c + 2]], buf, gsem)

        return carry

    lax.fori_loop(0, N_CHUNKS // 2, body, 0)
    pltpu.make_async_copy(
        buf0, out_hbm.at[pl.ds(base + (N_CHUNKS - 2) * CHUNK, CHUNK)], s0
    ).wait()
    pltpu.make_async_copy(
        buf1, out_hbm.at[pl.ds(base + (N_CHUNKS - 1) * CHUNK, CHUNK)], s1
    ).wait()


def kernel(index, cluster_index):
    idx = index.reshape(NW, N_CHUNKS, CHUNK)
    out = _gather_k(idx, cluster_index)
    return out.reshape(BATCH, N_FIELDS, EMBED_DIM)


# speedup vs baseline: 1.0821x; 1.0034x over previous
import functools

import jax
import jax.numpy as jnp
from jax import lax
from jax.experimental import pallas as pl
from jax.experimental.pallas import tpu as pltpu
from jax.experimental.pallas import tpu_sc as plsc

EMBED_DIM = 64
BATCH = 16384
N_FIELDS = 26
TOTAL = BATCH * N_FIELDS

NUM_CORES = 2
NUM_SUBCORES = 16
NW = NUM_CORES * NUM_SUBCORES
PER_W = TOTAL // NW
CHUNK = 512
N_CHUNKS = PER_W // CHUNK
KSUB = 4
SUB = CHUNK // KSUB

_mesh = plsc.VectorSubcoreMesh(core_axis_name="c", subcore_axis_name="s")


@functools.partial(
    pl.kernel,
    mesh=_mesh,
    out_type=jax.ShapeDtypeStruct((TOTAL, EMBED_DIM), jnp.float32),
    scratch_types=[
        pltpu.VMEM((N_CHUNKS * KSUB, SUB), jnp.int32),
        pltpu.VMEM((CHUNK, EMBED_DIM), jnp.float32),
        pltpu.VMEM((CHUNK, EMBED_DIM), jnp.float32),
        pltpu.SemaphoreType.DMA,
        pltpu.SemaphoreType.DMA,
        pltpu.SemaphoreType.DMA,
        pltpu.SemaphoreType.DMA,
    ],
    compiler_params=pltpu.CompilerParams(use_tc_tiling_on_sc=False),
)
def _gather_k(idx_hbm, table_hbm, out_hbm, idx_v, buf0, buf1, g0, g1, s0, s1):
    wid = lax.axis_index("s") * NUM_CORES + lax.axis_index("c")
    base = wid * PER_W
    pltpu.sync_copy(idx_hbm.at[wid], idx_v)

    bufs = (buf0, buf1)
    gsems = (g0, g1)
    ssems = (s0, s1)

    def start_gather(c, buf, gsem):
        for q in range(KSUB):
            pltpu.async_copy(
                table_hbm.at[idx_v.at[c * KSUB + q]],
                buf.at[pl.ds(q * SUB, SUB)],
                gsem,
            )

    def wait_gather(buf, gsem):
        pltpu.make_async_copy(table_hbm.at[pl.ds(0, CHUNK)], buf, gsem).wait()

    start_gather(0, buf0, g0)
    start_gather(1, buf1, g1)

    def body(i, carry):
        for p in range(2):
            c = i * 2 + p
            buf, gsem, ssem = bufs[p], gsems[p], ssems[p]
            wait_gather(buf, gsem)
            pltpu.async_copy(buf, out_hbm.at[pl.ds(base + c * CHUNK, CHUNK)], ssem)

            @pl.when(c + 2 < N_CHUNKS)
            def _():
                pltpu.make_async_copy(
                    buf, out_hbm.at[pl.ds(base + c * CHUNK, CHUNK)], ssem
                ).wait()
                start_gather(c + 2, buf, gsem)

        return carry

    lax.fori_loop(0, N_CHUNKS // 2, body, 0)
    pltpu.make_async_copy(
        buf0, out_hbm.at[pl.ds(base + (N_CHUNKS - 2) * CHUNK, CHUNK)], s0
    ).wait()
    pltpu.make_async_copy(
        buf1, out_hbm.at[pl.ds(base + (N_CHUNKS - 1) * CHUNK, CHUNK)], s1
    ).wait()


def kernel(index, cluster_index):
    idx = index.reshape(NW, N_CHUNKS * KSUB, SUB)
    out = _gather_k(idx, cluster_index)
    return out.reshape(BATCH, N_FIELDS, EMBED_DIM)
